# baseline (speedup 1.0000x reference)
import jax
import jax.numpy as jnp
from jax import lax
from jax.experimental import pallas as pl
from jax.experimental.pallas import tpu as pltpu
from jax.experimental.pallas import tpu_sc as plsc

N = 10000
N_PAD = 10032
IN_DIM = 128
EMB = 64
HID = 64
WN = 64
WD = 16
NC = 2
NS = 16
L = 16
NW = NC * NS
CHUNK = 128
E_REAL = 320000 + N
PER_TILE = 10368
EP = PER_TILE * NW
NCH = PER_TILE // CHUNK
RK = 16


def _pre_body(x_ref, w1_ref, b1_ref, wg_ref, att_s_ref, att_d_ref,
              hg_ref, asrc_ref, adst_ref, m_ref):
    x = x_ref[...]
    h = jnp.maximum(
        lax.dot_general(x, w1_ref[...], (((1,), (1,)), ((), ())),
                        preferred_element_type=jnp.float32) + b1_ref[...], 0.0)
    hg = lax.dot_general(h, wg_ref[...], (((1,), (1,)), ((), ())),
                         preferred_element_type=jnp.float32)
    hg_ref[...] = hg
    a_s = lax.dot_general(hg, att_s_ref[...], (((1,), (1,)), ((), ())),
                          preferred_element_type=jnp.float32)
    a_d = lax.dot_general(hg, att_d_ref[...], (((1,), (1,)), ((), ())),
                          preferred_element_type=jnp.float32)
    asrc_ref[...] = a_s
    adst_ref[...] = a_d
    mm = jnp.max(a_s) + jnp.max(a_d)
    m_ref[...] = jnp.broadcast_to(jnp.maximum(mm, 0.2 * mm), (1, 1))


def _pre(x, W1, b1, Wg, att_src, att_dst):
    return pl.pallas_call(
        _pre_body,
        out_shape=[
            jax.ShapeDtypeStruct((N, HID), jnp.float32),
            jax.ShapeDtypeStruct((N, 1), jnp.float32),
            jax.ShapeDtypeStruct((N, 1), jnp.float32),
            jax.ShapeDtypeStruct((1, 1), jnp.float32),
        ],
    )(x, W1, b1.reshape(1, EMB), Wg, att_src.reshape(1, HID),
      att_dst.reshape(1, HID))


def _stage_and_zero(asrc_hbm, adst_hbm, m_hbm, src_hbm, dst_hbm, wid, sid,
                    asrc_v, adst_v, mv, srcall, dstall, zbuf, acc, width):
    pltpu.sync_copy(asrc_hbm, asrc_v)
    pltpu.sync_copy(adst_hbm, adst_v)
    pltpu.sync_copy(m_hbm, mv)
    pltpu.sync_copy(src_hbm.at[wid], srcall)
    pltpu.sync_copy(dst_hbm.at[wid], dstall)
    for r in range(CHUNK):
        for c in range(width // L):
            zbuf[r, c * L:(c + 1) * L] = jnp.zeros((L,), jnp.float32)
    rpt = N_PAD // NS
    base0 = sid * rpt
    for k in range(rpt // CHUNK):
        pltpu.sync_copy(zbuf, acc.at[pl.ds(base0 + k * CHUNK, CHUNK)])
    rem = rpt % CHUNK
    if rem:
        pltpu.sync_copy(zbuf.at[pl.ds(0, rem)],
                        acc.at[pl.ds(base0 + (rpt // CHUNK) * CHUNK, rem)])
    plsc.subcore_barrier()
    return base0, rpt


def _make_w(wid, srcall, dstall, asrc_v, adst_v, mreg, sink):
    def compute_w(t):
        trow = jnp.full((L,), t, jnp.int32)

        def wgrp(g):
            lane = g * L + lax.iota(jnp.int32, L)
            si = plsc.load_gather(srcall, [trow, lane])
            di = plsc.load_gather(dstall, [trow, lane])
            a = plsc.load_gather(asrc_v, [si]) + plsc.load_gather(adst_v, [di])
            e = jnp.maximum(a, 0.2 * a)
            w = jnp.exp(e - mreg)
            gid = (wid * PER_TILE + t * CHUNK) + lane
            w = jnp.where(gid < E_REAL, w, 0.0)
            sink(lane, w)
        plsc.parallel_loop(0, CHUNK // L, step=1, unroll=CHUNK // L)(wgrp)
    return compute_w


def _num_body(hg_hbm, src_hbm, dst_hbm, asrc_hbm, adst_hbm, m_hbm, out_hbm,
              asrc_v, adst_v, mv, srcall, dstall, wbuf,
              rows0, rows1, ob0, ob1, acc, gsem0, gsem1, ssem0, ssem1):
    cid = lax.axis_index("c")
    sid = lax.axis_index("s")
    wid = cid * NS + sid
    base0, rpt = _stage_and_zero(asrc_hbm, adst_hbm, m_hbm, src_hbm, dst_hbm,
                                 wid, sid, asrc_v, adst_v, mv, srcall, dstall,
                                 rows0, acc, WN)
    mreg = mv[...]
    compute_w = _make_w(wid, srcall, dstall, asrc_v, adst_v, mreg,
                        lambda lane, w: plsc.store_scatter(wbuf, [lane], w))

    def start_gather(t, rows, gsem):
        pltpu.async_copy(hg_hbm.at[srcall.at[t]], rows, gsem)

    def wait_gather(t, rows, gsem):
        pltpu.make_async_copy(hg_hbm.at[srcall.at[t]], rows, gsem).wait()

    def start_scatter(t, ob, ssem):
        pltpu.async_copy(ob, acc.at[dstall.at[t]], ssem, add=True)

    def wait_scatter(t, ob, ssem):
        pltpu.make_async_copy(ob, acc.at[dstall.at[t]], ssem).wait()

    def scale(rows, ob):
        def row_body(r):
            ridx = jnp.full((L,), r, jnp.int32)
            wb = plsc.load_gather(wbuf, [ridx])
            for c in range(WN // L):
                cidx = c * L + lax.iota(jnp.int32, L)
                v = plsc.load_gather(rows, [ridx, cidx])
                plsc.store_scatter(ob, [ridx, cidx], v * wb)
        plsc.parallel_loop(0, CHUNK, step=1, unroll=RK)(row_body)

    def step(t, rows, ob, gsem, ssem):
        compute_w(t)
        wait_gather(t, rows, gsem)

        @pl.when(t >= 2)
        def _():
            wait_scatter(t - 2, ob, ssem)
        scale(rows, ob)

        @pl.when(t + 2 < NCH)
        def _():
            start_gather(t + 2, rows, gsem)
        start_scatter(t, ob, ssem)

    start_gather(0, rows0, gsem0)
    start_gather(1, rows1, gsem1)

    def pair(i, c2):
        step(i * 2, rows0, ob0, gsem0, ssem0)
        step(i * 2 + 1, rows1, ob1, gsem1, ssem1)
        return c2
    lax.fori_loop(0, NCH // 2, pair, 0)
    step(NCH - 1, rows0, ob0, gsem0, ssem0)
    wait_scatter(NCH - 2, ob1, ssem1)
    wait_scatter(NCH - 1, ob0, ssem0)
    plsc.subcore_barrier()
    pltpu.sync_copy(acc.at[pl.ds(base0, rpt)],
                    out_hbm.at[cid, pl.ds(base0, rpt)])


def _den_body(src_hbm, dst_hbm, asrc_hbm, adst_hbm, m_hbm, out_hbm,
              asrc_v, adst_v, mv, srcall, dstall, wr0, wr1, acc,
              ssem0, ssem1):
    cid = lax.axis_index("c")
    sid = lax.axis_index("s")
    wid = cid * NS + sid
    base0, rpt = _stage_and_zero(asrc_hbm, adst_hbm, m_hbm, src_hbm, dst_hbm,
                                 wid, sid, asrc_v, adst_v, mv, srcall, dstall,
                                 wr0, acc, WD)
    for r in range(CHUNK):
        wr1[r, 0:L] = jnp.zeros((L,), jnp.float32)
    mreg = mv[...]
    zcol = jnp.zeros((L,), jnp.int32)

    def start_scatter(t, wr, ssem):
        pltpu.async_copy(wr, acc.at[dstall.at[t]], ssem, add=True)

    def wait_scatter(t, wr, ssem):
        pltpu.make_async_copy(wr, acc.at[dstall.at[t]], ssem).wait()

    def step(t, wr, ssem):
        compute_w = _make_w(
            wid, srcall, dstall, asrc_v, adst_v, mreg,
            lambda lane, w: plsc.store_scatter(wr, [lane, zcol], w))

        @pl.when(t >= 2)
        def _():
            wait_scatter(t - 2, wr, ssem)
        compute_w(t)
        start_scatter(t, wr, ssem)

    def pair(i, c2):
        step(i * 2, wr0, ssem0)
        step(i * 2 + 1, wr1, ssem1)
        return c2
    lax.fori_loop(0, NCH // 2, pair, 0)
    step(NCH - 1, wr0, ssem0)
    wait_scatter(NCH - 2, wr1, ssem1)
    wait_scatter(NCH - 1, wr0, ssem0)
    plsc.subcore_barrier()
    pltpu.sync_copy(acc.at[pl.ds(base0, rpt)],
                    out_hbm.at[cid, pl.ds(base0, rpt)])


def _sc_mesh():
    return plsc.VectorSubcoreMesh(core_axis_name="c", subcore_axis_name="s",
                                  num_cores=NC, num_subcores=NS)


_SC_PARAMS = pltpu.CompilerParams(needs_layout_passes=False,
                                  use_tc_tiling_on_sc=False)


def _sc_num(hg, src3, dst3, a_src, a_dst, m16):
    k = pl.kernel(
        _num_body,
        out_type=[jax.ShapeDtypeStruct((NC, N_PAD, WN), jnp.float32)],
        mesh=_sc_mesh(),
        compiler_params=_SC_PARAMS,
        scratch_types=[
            pltpu.VMEM((N,), jnp.float32),
            pltpu.VMEM((N,), jnp.float32),
            pltpu.VMEM((L,), jnp.float32),
            pltpu.VMEM((NCH, CHUNK), jnp.int32),
            pltpu.VMEM((NCH, CHUNK), jnp.int32),
            pltpu.VMEM((CHUNK,), jnp.float32),
            pltpu.VMEM((CHUNK, WN), jnp.float32),
            pltpu.VMEM((CHUNK, WN), jnp.float32),
            pltpu.VMEM((CHUNK, WN), jnp.float32),
            pltpu.VMEM((CHUNK, WN), jnp.float32),
            pltpu.VMEM_SHARED((N_PAD, WN), jnp.float32),
            pltpu.SemaphoreType.DMA,
            pltpu.SemaphoreType.DMA,
            pltpu.SemaphoreType.DMA,
            pltpu.SemaphoreType.DMA,
        ],
    )
    (acc,) = k(hg, src3, dst3, a_src, a_dst, m16)
    return acc


def _sc_den(src3, dst3, a_src, a_dst, m16):
    k = pl.kernel(
        _den_body,
        out_type=[jax.ShapeDtypeStruct((NC, N_PAD, WD), jnp.float32)],
        mesh=_sc_mesh(),
        compiler_params=_SC_PARAMS,
        scratch_types=[
            pltpu.VMEM((N,), jnp.float32),
            pltpu.VMEM((N,), jnp.float32),
            pltpu.VMEM((L,), jnp.float32),
            pltpu.VMEM((NCH, CHUNK), jnp.int32),
            pltpu.VMEM((NCH, CHUNK), jnp.int32),
            pltpu.VMEM((CHUNK, WD), jnp.float32),
            pltpu.VMEM((CHUNK, WD), jnp.float32),
            pltpu.VMEM_SHARED((N_PAD, WD), jnp.float32),
            pltpu.SemaphoreType.DMA,
            pltpu.SemaphoreType.DMA,
        ],
    )
    (acc,) = k(src3, dst3, a_src, a_dst, m16)
    return acc


def _emb_body(num_ref, den_ref, bg_ref, emb_ref):
    num = num_ref[0][:N] + num_ref[1][:N]
    den = den_ref[0][:N, 0:1] + den_ref[1][:N, 0:1]
    emb_ref[...] = num / (den + 1e-16) + bg_ref[...]


def _emb(num, den, bg):
    return pl.pallas_call(
        _emb_body,
        out_shape=jax.ShapeDtypeStruct((N, HID), jnp.float32),
    )(num, den, bg.reshape(1, HID))


_SBLK = 128


def _s_body(embi_ref, embj_ref, out_ref):
    p = lax.dot_general(embi_ref[...], embj_ref[...],
                        (((1,), (1,)), ((), ())),
                        preferred_element_type=jnp.float32)
    out_ref[...] = 0.5 * jnp.tanh(0.5 * p) + 0.5


def _recon_s(emb):
    g = pl.cdiv(N, _SBLK)
    return pl.pallas_call(
        _s_body,
        grid=(g,),
        in_specs=[
            pl.BlockSpec((_SBLK, HID), lambda i: (i, 0)),
            pl.BlockSpec((N, HID), lambda i: (0, 0)),
        ],
        out_specs=pl.BlockSpec((_SBLK, N), lambda i: (i, 0)),
        out_shape=jax.ShapeDtypeStruct((N, N), jnp.float32),
    )(emb, emb)


def _attr_body(x_ref, w2_ref, b2_ref, w3_ref, b3_ref, emb_ref, out_ref):
    za = lax.dot_general(w2_ref[...], x_ref[...], (((1,), (0,)), ((), ())),
                         preferred_element_type=jnp.float32)
    t = jnp.maximum(za + b2_ref[...], 0.0)
    y = lax.dot_general(w3_ref[...], t, (((1,), (0,)), ((), ())),
                        preferred_element_type=jnp.float32) + b3_ref[...]
    out_ref[...] = lax.dot_general(emb_ref[...], y, (((1,), (0,)), ((), ())),
                                   preferred_element_type=jnp.float32)


def _attr(x, W2, b2, W3, b3, emb):
    return pl.pallas_call(
        _attr_body,
        out_shape=jax.ShapeDtypeStruct((N, IN_DIM), jnp.float32),
    )(x, W2, b2.reshape(EMB, 1), W3, b3.reshape(HID, 1), emb)


def kernel(x, edge_index, batch_size, W1, b1, Wg, att_src, att_dst, bg,
           W2, b2, W3, b3):
    hg, a_s, a_d, m = _pre(x, W1, b1, Wg, att_src, att_dst)

    loop = jnp.arange(N, dtype=jnp.int32)
    src = jnp.concatenate([edge_index[0].astype(jnp.int32), loop])
    dst = jnp.concatenate([edge_index[1].astype(jnp.int32), loop])
    src3 = jnp.pad(src, (0, EP - E_REAL)).reshape(NW, NCH, CHUNK)
    dst3 = jnp.pad(dst, (0, EP - E_REAL)).reshape(NW, NCH, CHUNK)
    m16 = jnp.broadcast_to(m.reshape(1), (L,))
    a_s = a_s.reshape(N)
    a_d = a_d.reshape(N)

    num = _sc_num(hg, src3, dst3, a_s, a_d, m16)
    den = _sc_den(src3, dst3, a_s, a_d, m16)
    emb = _emb(num, den, bg)
    s_ = _recon_s(emb)
    x_ = _attr(x, W2, b2, W3, b3, emb)
    return (x_, s_)


# speedup vs baseline: 1.0143x; 1.0143x over previous
import jax
import jax.numpy as jnp
from jax import lax
from jax.experimental import pallas as pl
from jax.experimental.pallas import tpu as pltpu
from jax.experimental.pallas import tpu_sc as plsc

N = 10000
N_PAD = 10032
IN_DIM = 128
EMB = 64
HID = 64
WN = 64
WD = 16
NC = 2
NS = 16
L = 16
NW = NC * NS
CHUNK = 128
E_REAL = 320000 + N
PER_TILE = 10368
EP = PER_TILE * NW
NCH = PER_TILE // CHUNK
RK = 16


def _pre_body(x_ref, w1_ref, b1_ref, wg_ref, att_s_ref, att_d_ref,
              hg_ref, asrc_ref, adst_ref, m_ref):
    x = x_ref[...]
    h = jnp.maximum(
        lax.dot_general(x, w1_ref[...], (((1,), (1,)), ((), ())),
                        preferred_element_type=jnp.float32) + b1_ref[...], 0.0)
    hg = lax.dot_general(h, wg_ref[...], (((1,), (1,)), ((), ())),
                         preferred_element_type=jnp.float32)
    hg_ref[...] = hg
    a_s = lax.dot_general(hg, att_s_ref[...], (((1,), (1,)), ((), ())),
                          preferred_element_type=jnp.float32)
    a_d = lax.dot_general(hg, att_d_ref[...], (((1,), (1,)), ((), ())),
                          preferred_element_type=jnp.float32)
    asrc_ref[...] = a_s
    adst_ref[...] = a_d
    mm = jnp.max(a_s) + jnp.max(a_d)
    m_ref[...] = jnp.broadcast_to(jnp.maximum(mm, 0.2 * mm), (1, 1))


def _pre(x, W1, b1, Wg, att_src, att_dst):
    return pl.pallas_call(
        _pre_body,
        out_shape=[
            jax.ShapeDtypeStruct((N, HID), jnp.float32),
            jax.ShapeDtypeStruct((N, 1), jnp.float32),
            jax.ShapeDtypeStruct((N, 1), jnp.float32),
            jax.ShapeDtypeStruct((1, 1), jnp.float32),
        ],
    )(x, W1, b1.reshape(1, EMB), Wg, att_src.reshape(1, HID),
      att_dst.reshape(1, HID))


def _stage_and_zero(asrc_hbm, adst_hbm, m_hbm, src_hbm, dst_hbm, wid, sid,
                    asrc_v, adst_v, mv, srcall, dstall, zbuf, acc, width):
    pltpu.sync_copy(asrc_hbm, asrc_v)
    pltpu.sync_copy(adst_hbm, adst_v)
    pltpu.sync_copy(m_hbm, mv)
    pltpu.sync_copy(src_hbm.at[wid], srcall)
    pltpu.sync_copy(dst_hbm.at[wid], dstall)
    for r in range(CHUNK):
        for c in range(width // L):
            zbuf[r, c * L:(c + 1) * L] = jnp.zeros((L,), jnp.float32)
    rpt = N_PAD // NS
    base0 = sid * rpt
    for k in range(rpt // CHUNK):
        pltpu.sync_copy(zbuf, acc.at[pl.ds(base0 + k * CHUNK, CHUNK)])
    rem = rpt % CHUNK
    if rem:
        pltpu.sync_copy(zbuf.at[pl.ds(0, rem)],
                        acc.at[pl.ds(base0 + (rpt // CHUNK) * CHUNK, rem)])
    plsc.subcore_barrier()
    return base0, rpt


def _make_w(wid, srcall, dstall, asrc_v, adst_v, mreg, sink):
    def compute_w(t):
        trow = jnp.full((L,), t, jnp.int32)

        def wgrp(g):
            lane = g * L + lax.iota(jnp.int32, L)
            si = plsc.load_gather(srcall, [trow, lane])
            di = plsc.load_gather(dstall, [trow, lane])
            a = plsc.load_gather(asrc_v, [si]) + plsc.load_gather(adst_v, [di])
            e = jnp.maximum(a, 0.2 * a)
            w = jnp.exp(e - mreg)
            gid = (wid * PER_TILE + t * CHUNK) + lane
            w = jnp.where(gid < E_REAL, w, 0.0)
            sink(lane, w)
        plsc.parallel_loop(0, CHUNK // L, step=1, unroll=CHUNK // L)(wgrp)
    return compute_w


def _num_body(hg_hbm, src_hbm, dst_hbm, asrc_hbm, adst_hbm, m_hbm, out_hbm,
              asrc_v, adst_v, mv, srcall, dstall, wbuf,
              rows0, rows1, ob0, ob1, acc, gsem0, gsem1, ssem0, ssem1):
    cid = lax.axis_index("c")
    sid = lax.axis_index("s")
    wid = cid * NS + sid
    base0, rpt = _stage_and_zero(asrc_hbm, adst_hbm, m_hbm, src_hbm, dst_hbm,
                                 wid, sid, asrc_v, adst_v, mv, srcall, dstall,
                                 rows0, acc, WN)
    mreg = mv[...]
    compute_w = _make_w(wid, srcall, dstall, asrc_v, adst_v, mreg,
                        lambda lane, w: plsc.store_scatter(wbuf, [lane], w))

    def start_gather(t, rows, gsem):
        pltpu.async_copy(hg_hbm.at[srcall.at[t]], rows, gsem)

    def wait_gather(t, rows, gsem):
        pltpu.make_async_copy(hg_hbm.at[srcall.at[t]], rows, gsem).wait()

    def start_scatter(t, ob, ssem):
        pltpu.async_copy(ob, acc.at[dstall.at[t]], ssem, add=True)

    def wait_scatter(t, ob, ssem):
        pltpu.make_async_copy(ob, acc.at[dstall.at[t]], ssem).wait()

    def scale(rows, ob):
        def row_body(r):
            ridx = jnp.full((L,), r, jnp.int32)
            wb = plsc.load_gather(wbuf, [ridx])
            for c in range(WN // L):
                v = rows[r, c * L:(c + 1) * L]
                ob[r, c * L:(c + 1) * L] = v * wb
        plsc.parallel_loop(0, CHUNK, step=1, unroll=RK)(row_body)

    def step(t, rows, ob, gsem, ssem):
        compute_w(t)
        wait_gather(t, rows, gsem)

        @pl.when(t >= 2)
        def _():
            wait_scatter(t - 2, ob, ssem)
        scale(rows, ob)

        @pl.when(t + 2 < NCH)
        def _():
            start_gather(t + 2, rows, gsem)
        start_scatter(t, ob, ssem)

    start_gather(0, rows0, gsem0)
    start_gather(1, rows1, gsem1)

    def pair(i, c2):
        step(i * 2, rows0, ob0, gsem0, ssem0)
        step(i * 2 + 1, rows1, ob1, gsem1, ssem1)
        return c2
    lax.fori_loop(0, NCH // 2, pair, 0)
    step(NCH - 1, rows0, ob0, gsem0, ssem0)
    wait_scatter(NCH - 2, ob1, ssem1)
    wait_scatter(NCH - 1, ob0, ssem0)
    plsc.subcore_barrier()
    pltpu.sync_copy(acc.at[pl.ds(base0, rpt)],
                    out_hbm.at[cid, pl.ds(base0, rpt)])


def _den_body(src_hbm, dst_hbm, asrc_hbm, adst_hbm, m_hbm, out_hbm,
              asrc_v, adst_v, mv, srcall, dstall, wr0, wr1, acc,
              ssem0, ssem1):
    cid = lax.axis_index("c")
    sid = lax.axis_index("s")
    wid = cid * NS + sid
    base0, rpt = _stage_and_zero(asrc_hbm, adst_hbm, m_hbm, src_hbm, dst_hbm,
                                 wid, sid, asrc_v, adst_v, mv, srcall, dstall,
                                 wr0, acc, WD)
    for r in range(CHUNK):
        wr1[r, 0:L] = jnp.zeros((L,), jnp.float32)
    mreg = mv[...]
    zcol = jnp.zeros((L,), jnp.int32)

    def start_scatter(t, wr, ssem):
        pltpu.async_copy(wr, acc.at[dstall.at[t]], ssem, add=True)

    def wait_scatter(t, wr, ssem):
        pltpu.make_async_copy(wr, acc.at[dstall.at[t]], ssem).wait()

    def step(t, wr, ssem):
        compute_w = _make_w(
            wid, srcall, dstall, asrc_v, adst_v, mreg,
            lambda lane, w: plsc.store_scatter(wr, [lane, zcol], w))

        @pl.when(t >= 2)
        def _():
            wait_scatter(t - 2, wr, ssem)
        compute_w(t)
        start_scatter(t, wr, ssem)

    def pair(i, c2):
        step(i * 2, wr0, ssem0)
        step(i * 2 + 1, wr1, ssem1)
        return c2
    lax.fori_loop(0, NCH // 2, pair, 0)
    step(NCH - 1, wr0, ssem0)
    wait_scatter(NCH - 2, wr1, ssem1)
    wait_scatter(NCH - 1, wr0, ssem0)
    plsc.subcore_barrier()
    pltpu.sync_copy(acc.at[pl.ds(base0, rpt)],
                    out_hbm.at[cid, pl.ds(base0, rpt)])


def _sc_mesh():
    return plsc.VectorSubcoreMesh(core_axis_name="c", subcore_axis_name="s",
                                  num_cores=NC, num_subcores=NS)


_SC_PARAMS = pltpu.CompilerParams(needs_layout_passes=False,
                                  use_tc_tiling_on_sc=False)


def _sc_num(hg, src3, dst3, a_src, a_dst, m16):
    k = pl.kernel(
        _num_body,
        out_type=[jax.ShapeDtypeStruct((NC, N_PAD, WN), jnp.float32)],
        mesh=_sc_mesh(),
        compiler_params=_SC_PARAMS,
        scratch_types=[
            pltpu.VMEM((N,), jnp.float32),
            pltpu.VMEM((N,), jnp.float32),
            pltpu.VMEM((L,), jnp.float32),
            pltpu.VMEM((NCH, CHUNK), jnp.int32),
            pltpu.VMEM((NCH, CHUNK), jnp.int32),
            pltpu.VMEM((CHUNK,), jnp.float32),
            pltpu.VMEM((CHUNK, WN), jnp.float32),
            pltpu.VMEM((CHUNK, WN), jnp.float32),
            pltpu.VMEM((CHUNK, WN), jnp.float32),
            pltpu.VMEM((CHUNK, WN), jnp.float32),
            pltpu.VMEM_SHARED((N_PAD, WN), jnp.float32),
            pltpu.SemaphoreType.DMA,
            pltpu.SemaphoreType.DMA,
            pltpu.SemaphoreType.DMA,
            pltpu.SemaphoreType.DMA,
        ],
    )
    (acc,) = k(hg, src3, dst3, a_src, a_dst, m16)
    return acc


def _sc_den(src3, dst3, a_src, a_dst, m16):
    k = pl.kernel(
        _den_body,
        out_type=[jax.ShapeDtypeStruct((NC, N_PAD, WD), jnp.float32)],
        mesh=_sc_mesh(),
        compiler_params=_SC_PARAMS,
        scratch_types=[
            pltpu.VMEM((N,), jnp.float32),
            pltpu.VMEM((N,), jnp.float32),
            pltpu.VMEM((L,), jnp.float32),
            pltpu.VMEM((NCH, CHUNK), jnp.int32),
            pltpu.VMEM((NCH, CHUNK), jnp.int32),
            pltpu.VMEM((CHUNK, WD), jnp.float32),
            pltpu.VMEM((CHUNK, WD), jnp.float32),
            pltpu.VMEM_SHARED((N_PAD, WD), jnp.float32),
            pltpu.SemaphoreType.DMA,
            pltpu.SemaphoreType.DMA,
        ],
    )
    (acc,) = k(src3, dst3, a_src, a_dst, m16)
    return acc


def _emb_body(num_ref, den_ref, bg_ref, emb_ref):
    num = num_ref[0][:N] + num_ref[1][:N]
    den = den_ref[0][:N, 0:1] + den_ref[1][:N, 0:1]
    emb_ref[...] = num / (den + 1e-16) + bg_ref[...]


def _emb(num, den, bg):
    return pl.pallas_call(
        _emb_body,
        out_shape=jax.ShapeDtypeStruct((N, HID), jnp.float32),
    )(num, den, bg.reshape(1, HID))


_SBLK = 256


def _s_body(embi_ref, embj_ref, out_ref):
    p = lax.dot_general(embi_ref[...], embj_ref[...],
                        (((1,), (1,)), ((), ())),
                        preferred_element_type=jnp.float32)
    out_ref[...] = 0.5 * jnp.tanh(0.5 * p) + 0.5


def _recon_s(emb):
    g = pl.cdiv(N, _SBLK)
    return pl.pallas_call(
        _s_body,
        grid=(g,),
        in_specs=[
            pl.BlockSpec((_SBLK, HID), lambda i: (i, 0)),
            pl.BlockSpec((N, HID), lambda i: (0, 0)),
        ],
        out_specs=pl.BlockSpec((_SBLK, N), lambda i: (i, 0)),
        out_shape=jax.ShapeDtypeStruct((N, N), jnp.float32),
    )(emb, emb)


def _attr_body(x_ref, w2_ref, b2_ref, w3_ref, b3_ref, emb_ref, out_ref):
    za = lax.dot_general(w2_ref[...], x_ref[...], (((1,), (0,)), ((), ())),
                         preferred_element_type=jnp.float32)
    t = jnp.maximum(za + b2_ref[...], 0.0)
    y = lax.dot_general(w3_ref[...], t, (((1,), (0,)), ((), ())),
                        preferred_element_type=jnp.float32) + b3_ref[...]
    out_ref[...] = lax.dot_general(emb_ref[...], y, (((1,), (0,)), ((), ())),
                                   preferred_element_type=jnp.float32)


def _attr(x, W2, b2, W3, b3, emb):
    return pl.pallas_call(
        _attr_body,
        out_shape=jax.ShapeDtypeStruct((N, IN_DIM), jnp.float32),
    )(x, W2, b2.reshape(EMB, 1), W3, b3.reshape(HID, 1), emb)


def kernel(x, edge_index, batch_size, W1, b1, Wg, att_src, att_dst, bg,
           W2, b2, W3, b3):
    hg, a_s, a_d, m = _pre(x, W1, b1, Wg, att_src, att_dst)

    loop = jnp.arange(N, dtype=jnp.int32)
    src = jnp.concatenate([edge_index[0].astype(jnp.int32), loop])
    dst = jnp.concatenate([edge_index[1].astype(jnp.int32), loop])
    src3 = jnp.pad(src, (0, EP - E_REAL)).reshape(NW, NCH, CHUNK)
    dst3 = jnp.pad(dst, (0, EP - E_REAL)).reshape(NW, NCH, CHUNK)
    m16 = jnp.broadcast_to(m.reshape(1), (L,))
    a_s = a_s.reshape(N)
    a_d = a_d.reshape(N)

    num = _sc_num(hg, src3, dst3, a_s, a_d, m16)
    den = _sc_den(src3, dst3, a_s, a_d, m16)
    emb = _emb(num, den, bg)
    s_ = _recon_s(emb)
    x_ = _attr(x, W2, b2, W3, b3, emb)
    return (x_, s_)


# speedup vs baseline: 1.0344x; 1.0198x over previous
import jax
import jax.numpy as jnp
from jax import lax
from jax.experimental import pallas as pl
from jax.experimental.pallas import tpu as pltpu
from jax.experimental.pallas import tpu_sc as plsc

N = 10000
N_PAD = 10032
IN_DIM = 128
EMB = 64
HID = 64
WN = 64
WD = 16
NC = 2
NS = 16
L = 16
NW = NC * NS
CHUNK = 128
E_REAL = 320000 + N
PER_TILE = 10368
EP = PER_TILE * NW
NCH = PER_TILE // CHUNK
RK = 16


def _pre_body(x_ref, w1_ref, b1_ref, wg_ref, att_s_ref, att_d_ref,
              hg_ref, asrc_ref, adst_ref, m_ref):
    x = x_ref[...]
    h = jnp.maximum(
        lax.dot_general(x, w1_ref[...], (((1,), (1,)), ((), ())),
                        preferred_element_type=jnp.float32) + b1_ref[...], 0.0)
    hg = lax.dot_general(h, wg_ref[...], (((1,), (1,)), ((), ())),
                         preferred_element_type=jnp.float32)
    hg_ref[...] = hg
    a_s = lax.dot_general(hg, att_s_ref[...], (((1,), (1,)), ((), ())),
                          preferred_element_type=jnp.float32)
    a_d = lax.dot_general(hg, att_d_ref[...], (((1,), (1,)), ((), ())),
                          preferred_element_type=jnp.float32)
    asrc_ref[...] = a_s
    adst_ref[...] = a_d
    mm = jnp.max(a_s) + jnp.max(a_d)
    m_ref[...] = jnp.broadcast_to(jnp.maximum(mm, 0.2 * mm), (1, 1))


def _pre(x, W1, b1, Wg, att_src, att_dst):
    return pl.pallas_call(
        _pre_body,
        out_shape=[
            jax.ShapeDtypeStruct((N, HID), jnp.float32),
            jax.ShapeDtypeStruct((N, 1), jnp.float32),
            jax.ShapeDtypeStruct((N, 1), jnp.float32),
            jax.ShapeDtypeStruct((1, 1), jnp.float32),
        ],
    )(x, W1, b1.reshape(1, EMB), Wg, att_src.reshape(1, HID),
      att_dst.reshape(1, HID))


def _stage_and_zero(asrc_hbm, adst_hbm, m_hbm, src_hbm, dst_hbm, wid, sid,
                    asrc_v, adst_v, mv, srcall, dstall, zbuf, acc, width):
    pltpu.sync_copy(asrc_hbm, asrc_v)
    pltpu.sync_copy(adst_hbm, adst_v)
    pltpu.sync_copy(m_hbm, mv)
    pltpu.sync_copy(src_hbm.at[wid], srcall)
    pltpu.sync_copy(dst_hbm.at[wid], dstall)
    for r in range(CHUNK):
        for c in range(width // L):
            zbuf[r, c * L:(c + 1) * L] = jnp.zeros((L,), jnp.float32)
    rpt = N_PAD // NS
    base0 = sid * rpt
    for k in range(rpt // CHUNK):
        pltpu.sync_copy(zbuf, acc.at[pl.ds(base0 + k * CHUNK, CHUNK)])
    rem = rpt % CHUNK
    if rem:
        pltpu.sync_copy(zbuf.at[pl.ds(0, rem)],
                        acc.at[pl.ds(base0 + (rpt // CHUNK) * CHUNK, rem)])
    plsc.subcore_barrier()
    return base0, rpt


def _make_w(wid, srcall, dstall, asrc_v, adst_v, mreg, sink):
    def compute_w(t):
        trow = jnp.full((L,), t, jnp.int32)

        def wgrp(g):
            lane = g * L + lax.iota(jnp.int32, L)
            si = plsc.load_gather(srcall, [trow, lane])
            di = plsc.load_gather(dstall, [trow, lane])
            a = plsc.load_gather(asrc_v, [si]) + plsc.load_gather(adst_v, [di])
            e = jnp.maximum(a, 0.2 * a)
            w = jnp.exp(e - mreg)
            gid = (t * NW + wid) * CHUNK + lane
            w = jnp.where(gid < E_REAL, w, 0.0)
            sink(lane, w)
        plsc.parallel_loop(0, CHUNK // L, step=1, unroll=CHUNK // L)(wgrp)
    return compute_w


def _num_body(hg_hbm, src_hbm, dst_hbm, asrc_hbm, adst_hbm, m_hbm, out_hbm,
              asrc_v, adst_v, mv, srcall, dstall, wbuf,
              rows0, rows1, ob0, ob1, acc, gsem0, gsem1, ssem0, ssem1):
    cid = lax.axis_index("c")
    sid = lax.axis_index("s")
    wid = cid * NS + sid
    base0, rpt = _stage_and_zero(asrc_hbm, adst_hbm, m_hbm, src_hbm, dst_hbm,
                                 wid, sid, asrc_v, adst_v, mv, srcall, dstall,
                                 rows0, acc, WN)
    mreg = mv[...]
    compute_w = _make_w(wid, srcall, dstall, asrc_v, adst_v, mreg,
                        lambda lane, w: plsc.store_scatter(wbuf, [lane], w))

    def start_gather(t, rows, gsem):
        pltpu.async_copy(hg_hbm.at[srcall.at[t]], rows, gsem)

    def wait_gather(t, rows, gsem):
        pltpu.make_async_copy(hg_hbm.at[srcall.at[t]], rows, gsem).wait()

    def start_scatter(t, ob, ssem):
        pltpu.async_copy(ob, acc.at[dstall.at[t]], ssem, add=True)

    def wait_scatter(t, ob, ssem):
        pltpu.make_async_copy(ob, acc.at[dstall.at[t]], ssem).wait()

    def scale(rows, ob):
        def row_body(r):
            ridx = jnp.full((L,), r, jnp.int32)
            wb = plsc.load_gather(wbuf, [ridx])
            for c in range(WN // L):
                cidx = c * L + lax.iota(jnp.int32, L)
                v = plsc.load_gather(rows, [ridx, cidx])
                plsc.store_scatter(ob, [ridx, cidx], v * wb)
        plsc.parallel_loop(0, CHUNK, step=1, unroll=RK)(row_body)

    def step(t, rows, ob, gsem, ssem):
        compute_w(t)
        wait_gather(t, rows, gsem)

        @pl.when(t >= 2)
        def _():
            wait_scatter(t - 2, ob, ssem)
        scale(rows, ob)

        @pl.when(t + 2 < NCH)
        def _():
            start_gather(t + 2, rows, gsem)
        start_scatter(t, ob, ssem)

    start_gather(0, rows0, gsem0)
    start_gather(1, rows1, gsem1)

    def pair(i, c2):
        step(i * 2, rows0, ob0, gsem0, ssem0)
        step(i * 2 + 1, rows1, ob1, gsem1, ssem1)
        return c2
    lax.fori_loop(0, NCH // 2, pair, 0)
    step(NCH - 1, rows0, ob0, gsem0, ssem0)
    wait_scatter(NCH - 2, ob1, ssem1)
    wait_scatter(NCH - 1, ob0, ssem0)
    plsc.subcore_barrier()
    pltpu.sync_copy(acc.at[pl.ds(base0, rpt)],
                    out_hbm.at[cid, pl.ds(base0, rpt)])


def _den_body(src_hbm, dst_hbm, asrc_hbm, adst_hbm, m_hbm, out_hbm,
              asrc_v, adst_v, mv, srcall, dstall, wr0, wr1, acc,
              ssem0, ssem1):
    cid = lax.axis_index("c")
    sid = lax.axis_index("s")
    wid = cid * NS + sid
    base0, rpt = _stage_and_zero(asrc_hbm, adst_hbm, m_hbm, src_hbm, dst_hbm,
                                 wid, sid, asrc_v, adst_v, mv, srcall, dstall,
                                 wr0, acc, WD)
    for r in range(CHUNK):
        wr1[r, 0:L] = jnp.zeros((L,), jnp.float32)
    mreg = mv[...]
    zcol = jnp.zeros((L,), jnp.int32)

    def start_scatter(t, wr, ssem):
        pltpu.async_copy(wr, acc.at[dstall.at[t]], ssem, add=True)

    def wait_scatter(t, wr, ssem):
        pltpu.make_async_copy(wr, acc.at[dstall.at[t]], ssem).wait()

    def step(t, wr, ssem):
        compute_w = _make_w(
            wid, srcall, dstall, asrc_v, adst_v, mreg,
            lambda lane, w: plsc.store_scatter(wr, [lane, zcol], w))

        @pl.when(t >= 2)
        def _():
            wait_scatter(t - 2, wr, ssem)
        compute_w(t)
        start_scatter(t, wr, ssem)

    def pair(i, c2):
        step(i * 2, wr0, ssem0)
        step(i * 2 + 1, wr1, ssem1)
        return c2
    lax.fori_loop(0, NCH // 2, pair, 0)
    step(NCH - 1, wr0, ssem0)
    wait_scatter(NCH - 2, wr1, ssem1)
    wait_scatter(NCH - 1, wr0, ssem0)
    plsc.subcore_barrier()
    pltpu.sync_copy(acc.at[pl.ds(base0, rpt)],
                    out_hbm.at[cid, pl.ds(base0, rpt)])


def _sc_mesh():
    return plsc.VectorSubcoreMesh(core_axis_name="c", subcore_axis_name="s",
                                  num_cores=NC, num_subcores=NS)


_SC_PARAMS = pltpu.CompilerParams(needs_layout_passes=False,
                                  use_tc_tiling_on_sc=False)


def _sc_num(hg, src3, dst3, a_src, a_dst, m16):
    k = pl.kernel(
        _num_body,
        out_type=[jax.ShapeDtypeStruct((NC, N_PAD, WN), jnp.float32)],
        mesh=_sc_mesh(),
        compiler_params=_SC_PARAMS,
        scratch_types=[
            pltpu.VMEM((N,), jnp.float32),
            pltpu.VMEM((N,), jnp.float32),
            pltpu.VMEM((L,), jnp.float32),
            pltpu.VMEM((NCH, CHUNK), jnp.int32),
            pltpu.VMEM((NCH, CHUNK), jnp.int32),
            pltpu.VMEM((CHUNK,), jnp.float32),
            pltpu.VMEM((CHUNK, WN), jnp.float32),
            pltpu.VMEM((CHUNK, WN), jnp.float32),
            pltpu.VMEM((CHUNK, WN), jnp.float32),
            pltpu.VMEM((CHUNK, WN), jnp.float32),
            pltpu.VMEM_SHARED((N_PAD, WN), jnp.float32),
            pltpu.SemaphoreType.DMA,
            pltpu.SemaphoreType.DMA,
            pltpu.SemaphoreType.DMA,
            pltpu.SemaphoreType.DMA,
        ],
    )
    (acc,) = k(hg, src3, dst3, a_src, a_dst, m16)
    return acc


def _sc_den(src3, dst3, a_src, a_dst, m16):
    k = pl.kernel(
        _den_body,
        out_type=[jax.ShapeDtypeStruct((NC, N_PAD, WD), jnp.float32)],
        mesh=_sc_mesh(),
        compiler_params=_SC_PARAMS,
        scratch_types=[
            pltpu.VMEM((N,), jnp.float32),
            pltpu.VMEM((N,), jnp.float32),
            pltpu.VMEM((L,), jnp.float32),
            pltpu.VMEM((NCH, CHUNK), jnp.int32),
            pltpu.VMEM((NCH, CHUNK), jnp.int32),
            pltpu.VMEM((CHUNK, WD), jnp.float32),
            pltpu.VMEM((CHUNK, WD), jnp.float32),
            pltpu.VMEM_SHARED((N_PAD, WD), jnp.float32),
            pltpu.SemaphoreType.DMA,
            pltpu.SemaphoreType.DMA,
        ],
    )
    (acc,) = k(src3, dst3, a_src, a_dst, m16)
    return acc


def _emb_body(num_ref, den_ref, bg_ref, emb_ref):
    num = num_ref[0][:N] + num_ref[1][:N]
    den = den_ref[0][:N, 0:1] + den_ref[1][:N, 0:1]
    emb_ref[...] = num / (den + 1e-16) + bg_ref[...]


def _emb(num, den, bg):
    return pl.pallas_call(
        _emb_body,
        out_shape=jax.ShapeDtypeStruct((N, HID), jnp.float32),
    )(num, den, bg.reshape(1, HID))


_SBLK = 256


def _s_body(embi_ref, embj_ref, out_ref):
    p = lax.dot_general(embi_ref[...], embj_ref[...],
                        (((1,), (1,)), ((), ())),
                        preferred_element_type=jnp.float32)
    out_ref[...] = 0.5 * jnp.tanh(0.5 * p) + 0.5


def _recon_s(emb):
    g = pl.cdiv(N, _SBLK)
    return pl.pallas_call(
        _s_body,
        grid=(g,),
        in_specs=[
            pl.BlockSpec((_SBLK, HID), lambda i: (i, 0)),
            pl.BlockSpec((N, HID), lambda i: (0, 0)),
        ],
        out_specs=pl.BlockSpec((_SBLK, N), lambda i: (i, 0)),
        out_shape=jax.ShapeDtypeStruct((N, N), jnp.float32),
    )(emb, emb)


def _attr_body(x_ref, w2_ref, b2_ref, w3_ref, b3_ref, emb_ref, out_ref):
    za = lax.dot_general(w2_ref[...], x_ref[...], (((1,), (0,)), ((), ())),
                         preferred_element_type=jnp.float32)
    t = jnp.maximum(za + b2_ref[...], 0.0)
    y = lax.dot_general(w3_ref[...], t, (((1,), (0,)), ((), ())),
                        preferred_element_type=jnp.float32) + b3_ref[...]
    out_ref[...] = lax.dot_general(emb_ref[...], y, (((1,), (0,)), ((), ())),
                                   preferred_element_type=jnp.float32)


def _attr(x, W2, b2, W3, b3, emb):
    return pl.pallas_call(
        _attr_body,
        out_shape=jax.ShapeDtypeStruct((N, IN_DIM), jnp.float32),
    )(x, W2, b2.reshape(EMB, 1), W3, b3.reshape(HID, 1), emb)


def kernel(x, edge_index, batch_size, W1, b1, Wg, att_src, att_dst, bg,
           W2, b2, W3, b3):
    hg, a_s, a_d, m = _pre(x, W1, b1, Wg, att_src, att_dst)

    loop = jnp.arange(N, dtype=jnp.int32)
    src = jnp.concatenate([edge_index[0].astype(jnp.int32), loop])
    dst = jnp.concatenate([edge_index[1].astype(jnp.int32), loop])
    src3 = jnp.pad(src, (0, EP - E_REAL)).reshape(NCH, NW, CHUNK)
    src3 = jnp.swapaxes(src3, 0, 1)
    dst3 = jnp.pad(dst, (0, EP - E_REAL)).reshape(NCH, NW, CHUNK)
    dst3 = jnp.swapaxes(dst3, 0, 1)
    m16 = jnp.broadcast_to(m.reshape(1), (L,))
    a_s = a_s.reshape(N)
    a_d = a_d.reshape(N)

    num = _sc_num(hg, src3, dst3, a_s, a_d, m16)
    den = _sc_den(src3, dst3, a_s, a_d, m16)
    emb = _emb(num, den, bg)
    s_ = _recon_s(emb)
    x_ = _attr(x, W2, b2, W3, b3, emb)
    return (x_, s_)
